# Initial kernel scaffold; baseline (speedup 1.0000x reference)
import jax
import jax.numpy as jnp
from jax.experimental import pallas as pl


def kernel(x, gate_w, gate_b, noise_w, noise_b):
    raise NotImplementedError("write your pallas kernel here")


# speedup vs baseline: 1.6243x; 1.6243x over previous
import functools

import jax
import jax.numpy as jnp
from jax import lax
from jax.experimental import pallas as pl
from jax.experimental.pallas import tpu as pltpu
from jax.experimental.pallas import tpu_sc as plsc

N_TOKENS = 8192
D_MODEL = 4096
N_EXPERTS = 64

L = 16
NW = 32
TPW = N_TOKENS // NW

TOK_BLK = 512


def _logits_body(x_ref, w_ref, b_ref, out_ref):
    out_ref[...] = (
        jnp.dot(x_ref[...], w_ref[...], preferred_element_type=jnp.float32)
        + b_ref[...]
    )


def _logits_tc(x, gate_w, gate_b):
    return pl.pallas_call(
        _logits_body,
        grid=(N_TOKENS // TOK_BLK,),
        in_specs=[
            pl.BlockSpec((TOK_BLK, D_MODEL), lambda i: (i, 0)),
            pl.BlockSpec((D_MODEL, N_EXPERTS), lambda i: (0, 0)),
            pl.BlockSpec((1, N_EXPERTS), lambda i: (0, 0)),
        ],
        out_specs=pl.BlockSpec((TOK_BLK, N_EXPERTS), lambda i: (i, 0)),
        out_shape=jax.ShapeDtypeStruct((N_TOKENS, N_EXPERTS), jnp.float32),
    )(x, gate_w, gate_b.reshape(1, N_EXPERTS))


@functools.partial(
    pl.kernel,
    out_type=(
        jax.ShapeDtypeStruct((N_TOKENS * N_EXPERTS,), jnp.float32),
        jax.ShapeDtypeStruct((N_TOKENS,), jnp.int32),
        jax.ShapeDtypeStruct((N_TOKENS,), jnp.int32),
    ),
    mesh=plsc.VectorSubcoreMesh(core_axis_name="c", subcore_axis_name="s"),
    compiler_params=pltpu.CompilerParams(needs_layout_passes=False),
    scratch_types=[
        pltpu.VMEM((TPW * N_EXPERTS,), jnp.float32),
        pltpu.VMEM((TPW * N_EXPERTS,), jnp.float32),
        pltpu.VMEM((TPW,), jnp.int32),
        pltpu.VMEM((TPW,), jnp.int32),
    ],
)
def _route_sc(lg_hbm, outr_hbm, i1_hbm, i2_hbm, lg_v, rout_v, i1_v, i2_v):
    nc = 2
    wid = lax.axis_index("s") * nc + lax.axis_index("c")
    base = wid * TPW
    pltpu.sync_copy(lg_hbm.at[pl.ds(base * N_EXPERTS, TPW * N_EXPERTS)], lg_v)

    zeros16 = jnp.zeros((L,), jnp.float32)

    def zbody(k, carry):
        rout_v[pl.ds(k * L, L)] = zeros16
        return carry

    lax.fori_loop(0, TPW * N_EXPERTS // L, zbody, 0)

    iota = lax.iota(jnp.int32, L)
    neg = jnp.full((L,), -jnp.inf, jnp.float32)
    zidx = jnp.zeros((L,), jnp.int32)

    def gbody(g, carry):
        t0 = g * L

        def ebody(e, tk):
            m1, m2, i1, i2 = tk
            v = plsc.load_gather(lg_v, [(t0 + iota) * N_EXPERTS + e])
            ev = jnp.broadcast_to(e, (L,)).astype(jnp.int32)
            gt1 = v > m1
            gt2 = jnp.logical_and(v > m2, jnp.logical_not(gt1))
            i2n = jnp.where(gt1, i1, jnp.where(gt2, ev, i2))
            m2n = jnp.where(gt1, m1, jnp.where(gt2, v, m2))
            i1n = jnp.where(gt1, ev, i1)
            m1n = jnp.where(gt1, v, m1)
            return (m1n, m2n, i1n, i2n)

        m1, m2, i1, i2 = lax.fori_loop(0, N_EXPERTS, ebody, (neg, neg, zidx, zidx))

        e2 = jnp.exp(m2 - m1)
        den = e2 + 1.0
        p1 = 1.0 / den
        p2 = e2 / den
        tflat = (t0 + iota) * N_EXPERTS
        plsc.store_scatter(rout_v, [tflat + i1], p1)
        plsc.store_scatter(rout_v, [tflat + i2], p2)
        i1_v[pl.ds(t0, L)] = i1
        i2_v[pl.ds(t0, L)] = i2
        return carry

    lax.fori_loop(0, TPW // L, gbody, 0)

    pltpu.sync_copy(rout_v, outr_hbm.at[pl.ds(base * N_EXPERTS, TPW * N_EXPERTS)])
    pltpu.sync_copy(i1_v, i1_hbm.at[pl.ds(base, TPW)])
    pltpu.sync_copy(i2_v, i2_hbm.at[pl.ds(base, TPW)])


def kernel(x, gate_w, gate_b, noise_w, noise_b):
    logits = _logits_tc(x, gate_w, gate_b)
    rout_flat, i1, i2 = _route_sc(logits.reshape(-1))
    router_output = rout_flat.reshape(N_TOKENS, N_EXPERTS)
    indices = jnp.stack([i1, i2], axis=-1)
    return (router_output, indices)


# speedup vs baseline: 1.9897x; 1.2249x over previous
import functools

import jax
import jax.numpy as jnp
from jax import lax
from jax.experimental import pallas as pl
from jax.experimental.pallas import tpu as pltpu
from jax.experimental.pallas import tpu_sc as plsc

N_TOKENS = 8192
D_MODEL = 4096
N_EXPERTS = 64

L = 16
NW = 32
TPW = N_TOKENS // NW

TOK_BLK = 512


def _logits_body(x_ref, w_ref, b_ref, out_ref):
    lg = (
        jnp.dot(x_ref[...], w_ref[...], preferred_element_type=jnp.float32)
        + b_ref[...]
    )
    out_ref[...] = lg.T


def _logits_tc(x, gate_w, gate_b):
    return pl.pallas_call(
        _logits_body,
        grid=(N_TOKENS // TOK_BLK,),
        in_specs=[
            pl.BlockSpec((TOK_BLK, D_MODEL), lambda i: (i, 0)),
            pl.BlockSpec((D_MODEL, N_EXPERTS), lambda i: (0, 0)),
            pl.BlockSpec((1, N_EXPERTS), lambda i: (0, 0)),
        ],
        out_specs=pl.BlockSpec((N_EXPERTS, TOK_BLK), lambda i: (0, i)),
        out_shape=jax.ShapeDtypeStruct((N_EXPERTS, N_TOKENS), jnp.float32),
    )(x, gate_w, gate_b.reshape(1, N_EXPERTS))


@functools.partial(
    pl.kernel,
    out_type=(
        jax.ShapeDtypeStruct((N_TOKENS * N_EXPERTS,), jnp.float32),
        jax.ShapeDtypeStruct((N_TOKENS,), jnp.int32),
        jax.ShapeDtypeStruct((N_TOKENS,), jnp.int32),
    ),
    mesh=plsc.VectorSubcoreMesh(core_axis_name="c", subcore_axis_name="s"),
    compiler_params=pltpu.CompilerParams(needs_layout_passes=False),
    scratch_types=[
        pltpu.VMEM((N_EXPERTS, TPW), jnp.float32),
        pltpu.VMEM((TPW * N_EXPERTS,), jnp.float32),
        pltpu.VMEM((TPW,), jnp.int32),
        pltpu.VMEM((TPW,), jnp.int32),
    ],
)
def _route_sc(lgt_hbm, outr_hbm, i1_hbm, i2_hbm, lg_v, rout_v, i1_v, i2_v):
    nc = 2
    wid = lax.axis_index("s") * nc + lax.axis_index("c")
    base = wid * TPW
    pltpu.sync_copy(lgt_hbm.at[:, pl.ds(base, TPW)], lg_v)

    iota = lax.iota(jnp.int32, L)
    neg = jnp.full((L,), -jnp.inf, jnp.float32)
    zidx = jnp.zeros((L,), jnp.int32)
    zeros16 = jnp.zeros((L,), jnp.float32)

    def gbody(g, carry):
        t0 = g * L
        for j in range(L * N_EXPERTS // L):
            rout_v[pl.ds(t0 * N_EXPERTS + j * L, L)] = zeros16

        m1, m2, i1, i2 = neg, neg, zidx, zidx
        for e in range(N_EXPERTS):
            v = lg_v[e, pl.ds(t0, L)]
            ev = jnp.full((L,), e, jnp.int32)
            gt1 = v > m1
            gt2 = jnp.logical_and(v > m2, jnp.logical_not(gt1))
            i2 = jnp.where(gt1, i1, jnp.where(gt2, ev, i2))
            m2 = jnp.where(gt1, m1, jnp.where(gt2, v, m2))
            i1 = jnp.where(gt1, ev, i1)
            m1 = jnp.where(gt1, v, m1)

        e2 = jnp.exp(m2 - m1)
        den = e2 + 1.0
        p1 = 1.0 / den
        p2 = e2 / den
        tflat = (t0 + iota) * N_EXPERTS
        plsc.store_scatter(rout_v, [tflat + i1], p1)
        plsc.store_scatter(rout_v, [tflat + i2], p2)
        i1_v[pl.ds(t0, L)] = i1
        i2_v[pl.ds(t0, L)] = i2
        return carry

    lax.fori_loop(0, TPW // L, gbody, 0)

    pltpu.sync_copy(rout_v, outr_hbm.at[pl.ds(base * N_EXPERTS, TPW * N_EXPERTS)])
    pltpu.sync_copy(i1_v, i1_hbm.at[pl.ds(base, TPW)])
    pltpu.sync_copy(i2_v, i2_hbm.at[pl.ds(base, TPW)])


def kernel(x, gate_w, gate_b, noise_w, noise_b):
    logits_t = _logits_tc(x, gate_w, gate_b)
    rout_flat, i1, i2 = _route_sc(logits_t)
    router_output = rout_flat.reshape(N_TOKENS, N_EXPERTS)
    indices = jnp.stack([i1, i2], axis=-1)
    return (router_output, indices)
